# Initial kernel scaffold; baseline (speedup 1.0000x reference)
import functools

import jax
import jax.numpy as jnp
from jax import lax
from jax.experimental import pallas as pl
from jax.experimental.pallas import tpu as pltpu
from jax.experimental.pallas import tpu_sc as plsc

N = 10000
E = 320000
D = 128
H = 64
EIN = 16
NC = 2
NS = 16
K = 256
KI = 128
EP = 327680
PER_TILE = EP // NS
CHUNKS = PER_TILE // K
NPT = N // NS

_LN2 = 0.6931471805599453


def _ssp(x):
    return jax.nn.softplus(x) - _LN2


def _hv_body(x_ref, w_ref, b_ref, out_ref):
    hv = jnp.dot(x_ref[...], w_ref[...], preferred_element_type=jnp.float32)
    hv = hv + b_ref[...]
    out_ref[0] = hv[:, :H]
    out_ref[1] = hv[:, H:]


def _hv_halves(x, w, b):
    BN = 2000
    return pl.pallas_call(
        _hv_body,
        grid=(N // BN,),
        in_specs=[
            pl.BlockSpec((BN, D), lambda i: (i, 0)),
            pl.BlockSpec((D, D), lambda i: (0, 0)),
            pl.BlockSpec((1, D), lambda i: (0, 0)),
        ],
        out_specs=pl.BlockSpec((2, BN, H), lambda i: (0, i, 0)),
        out_shape=jax.ShapeDtypeStruct((2, N, H), jnp.float32),
    )(x, w, b.reshape(1, D))


def _he_body(ef_ref, w1_ref, b1_ref, w2_ref, b2_ref, out_ref, *, be):
    h1 = jnp.dot(ef_ref[...], w1_ref[...], preferred_element_type=jnp.float32)
    h1 = _ssp(h1 + b1_ref[...])
    h2 = jnp.dot(h1, w2_ref[...], preferred_element_type=jnp.float32)
    h2 = _ssp(h2 + b2_ref[...])
    i = pl.program_id(0)
    row = i * be + lax.broadcasted_iota(jnp.int32, (be, 1), 0)
    h2 = jnp.where(row < E, h2, 0.0)
    out_ref[0] = h2[:, :H]
    out_ref[1] = h2[:, H:]


def _he_halves(ef, w1, b1, w2, b2):
    BE = 2048
    last = E // BE
    return pl.pallas_call(
        functools.partial(_he_body, be=BE),
        grid=(EP // BE,),
        in_specs=[
            pl.BlockSpec((BE, EIN), lambda i: (jnp.minimum(i, last), 0)),
            pl.BlockSpec((EIN, D), lambda i: (0, 0)),
            pl.BlockSpec((1, D), lambda i: (0, 0)),
            pl.BlockSpec((D, D), lambda i: (0, 0)),
            pl.BlockSpec((1, D), lambda i: (0, 0)),
        ],
        out_specs=pl.BlockSpec((2, BE, H), lambda i: (0, i, 0)),
        out_shape=jax.ShapeDtypeStruct((2, EP, H), jnp.float32),
    )(ef, w1, b1.reshape(1, D), w2, b2.reshape(1, D))


def _sc_body(hv_hbm, he_hbm, src_hbm, dst_hbm, zero_hbm, out_hbm,
             hv_s, acc_s, idx_src, idx_dst, rows, heb):
    c = lax.axis_index("c")
    s = lax.axis_index("s")

    nsl = pl.ds(s * NPT, NPT)
    pltpu.sync_copy(hv_hbm.at[c, nsl, :], hv_s.at[nsl, :])
    pltpu.sync_copy(zero_hbm, acc_s.at[nsl, :])
    plsc.subcore_barrier()

    base = s * PER_TILE

    def chunk(ci, carry):
        off = base + ci * K
        pltpu.sync_copy(src_hbm.at[pl.ds(off, K)], idx_src)
        pltpu.sync_copy(dst_hbm.at[pl.ds(off, K)], idx_dst)
        pltpu.sync_copy(he_hbm.at[c, pl.ds(off, K), :], heb)
        for q in range(K // KI):
            pltpu.sync_copy(hv_s.at[idx_src.at[q]],
                            rows.at[pl.ds(q * KI, KI), :])

        def mul_row(j, cc):
            for v in range(H // 16):
                sl = pl.ds(v * 16, 16)
                rows[j, sl] = rows[j, sl] * heb[j, sl]
            return cc
        lax.fori_loop(0, K, mul_row, 0)

        for q in range(K // KI):
            pltpu.sync_copy(rows.at[pl.ds(q * KI, KI), :],
                            acc_s.at[idx_dst.at[q]], add=True)
        return carry

    lax.fori_loop(0, CHUNKS, chunk, 0)
    plsc.subcore_barrier()

    pltpu.sync_copy(acc_s.at[nsl, :], out_hbm.at[c, nsl, :])


def _sc_aggregate(hv_halves, he_halves, src, dst, zeros):
    mesh = plsc.VectorSubcoreMesh(core_axis_name="c", subcore_axis_name="s")
    f = pl.kernel(
        _sc_body,
        out_type=jax.ShapeDtypeStruct((2, N, H), jnp.float32),
        mesh=mesh,
        scratch_types=[
            pltpu.VMEM_SHARED((N, H), jnp.float32),
            pltpu.VMEM_SHARED((N, H), jnp.float32),
            pltpu.VMEM((K // KI, KI), jnp.int32),
            pltpu.VMEM((K // KI, KI), jnp.int32),
            pltpu.VMEM((K, H), jnp.float32),
            pltpu.VMEM((K, H), jnp.float32),
        ],
    )
    return f(hv_halves, he_halves, src, dst, zeros)


def _out_body(hl_ref, hh_ref, wcl_ref, wch_ref, bc_ref, wo_ref, bo_ref,
              out_ref):
    t = jnp.dot(hl_ref[...], wcl_ref[...], preferred_element_type=jnp.float32)
    t = t + jnp.dot(hh_ref[...], wch_ref[...],
                    preferred_element_type=jnp.float32)
    t = _ssp(t + bc_ref[...])
    o = jnp.dot(t, wo_ref[...], preferred_element_type=jnp.float32)
    out_ref[...] = o + bo_ref[...]


def _project_out(h_halves, w_co, b_co, w_out, b_out):
    BN = 2000
    return pl.pallas_call(
        _out_body,
        grid=(N // BN,),
        in_specs=[
            pl.BlockSpec((BN, H), lambda i: (i, 0)),
            pl.BlockSpec((BN, H), lambda i: (i, 0)),
            pl.BlockSpec((H, D), lambda i: (0, 0)),
            pl.BlockSpec((H, D), lambda i: (0, 0)),
            pl.BlockSpec((1, D), lambda i: (0, 0)),
            pl.BlockSpec((D, D), lambda i: (0, 0)),
            pl.BlockSpec((1, D), lambda i: (0, 0)),
        ],
        out_specs=pl.BlockSpec((BN, D), lambda i: (i, 0)),
        out_shape=jax.ShapeDtypeStruct((N, D), jnp.float32),
    )(h_halves[0], h_halves[1], w_co[:H], w_co[H:], b_co.reshape(1, D),
      w_out, b_out.reshape(1, D))


def kernel(node_feats, edge_feats, edge_index,
           W_pn, b_pn, W_e1, b_e1, W_e2, b_e2, W_co, b_co, W_out, b_out):
    src = edge_index[0].astype(jnp.int32)
    dst = edge_index[1].astype(jnp.int32)
    pad = jnp.zeros((EP - E,), jnp.int32)
    src_p = jnp.concatenate([src, pad])
    dst_p = jnp.concatenate([dst, pad])
    zeros = jnp.zeros((NPT, H), jnp.float32)

    hv_halves = _hv_halves(node_feats, W_pn, b_pn)
    he_halves = _he_halves(edge_feats, W_e1, b_e1, W_e2, b_e2)
    h_halves = _sc_aggregate(hv_halves, he_halves, src_p, dst_p, zeros)
    return _project_out(h_halves, W_co, b_co, W_out, b_out)


# speedup vs baseline: 1.6471x; 1.6471x over previous
import functools

import jax
import jax.numpy as jnp
from jax import lax
from jax.experimental import pallas as pl
from jax.experimental.pallas import tpu as pltpu
from jax.experimental.pallas import tpu_sc as plsc

N = 10000
NP = 10240
E = 320000
D = 128
H = 64
EIN = 16
NC = 2
NS = 16
K = 128
KI = 128
GK = 8
EP = 327680
PER_TILE = EP // NS
GROUPS = PER_TILE // (GK * K)
NPT = NP // NS

_LN2 = 0.6931471805599453


def _ssp(x):
    return jax.nn.softplus(x) - _LN2


def _hv_body(x_ref, w_ref, b_ref, out_ref):
    hv = jnp.dot(x_ref[...], w_ref[...], preferred_element_type=jnp.float32)
    hv = hv + b_ref[...]
    out_ref[0] = hv[:, :H]
    out_ref[1] = hv[:, H:]


def _hv_halves(x, w, b):
    BN = 2048
    return pl.pallas_call(
        _hv_body,
        grid=(NP // BN,),
        in_specs=[
            pl.BlockSpec((BN, D), lambda i: (i, 0)),
            pl.BlockSpec((D, D), lambda i: (0, 0)),
            pl.BlockSpec((1, D), lambda i: (0, 0)),
        ],
        out_specs=pl.BlockSpec((2, BN, H), lambda i: (0, i, 0)),
        out_shape=jax.ShapeDtypeStruct((2, NP, H), jnp.float32),
    )(x, w, b.reshape(1, D))


def _he_body(ef_ref, w1_ref, b1_ref, w2_ref, b2_ref, out_ref, *, be):
    h1 = jnp.dot(ef_ref[...], w1_ref[...], preferred_element_type=jnp.float32)
    h1 = _ssp(h1 + b1_ref[...])
    h2 = jnp.dot(h1, w2_ref[...], preferred_element_type=jnp.float32)
    h2 = _ssp(h2 + b2_ref[...])
    i = pl.program_id(0)
    row = i * be + lax.broadcasted_iota(jnp.int32, (be, 1), 0)
    h2 = jnp.where(row < E, h2, 0.0)
    out_ref[0] = h2[:, :H]
    out_ref[1] = h2[:, H:]


def _he_halves(ef, w1, b1, w2, b2):
    BE = 2048
    last = E // BE
    return pl.pallas_call(
        functools.partial(_he_body, be=BE),
        grid=(EP // BE,),
        in_specs=[
            pl.BlockSpec((BE, EIN), lambda i: (jnp.minimum(i, last), 0)),
            pl.BlockSpec((EIN, D), lambda i: (0, 0)),
            pl.BlockSpec((1, D), lambda i: (0, 0)),
            pl.BlockSpec((D, D), lambda i: (0, 0)),
            pl.BlockSpec((1, D), lambda i: (0, 0)),
        ],
        out_specs=pl.BlockSpec((2, BE, H), lambda i: (0, i, 0)),
        out_shape=jax.ShapeDtypeStruct((2, EP, H), jnp.float32),
    )(ef, w1, b1.reshape(1, D), w2, b2.reshape(1, D))


def _sc_body(hv_hbm, he_hbm, src_hbm, dst_hbm, zero_hbm, out_hbm,
             hv_s, acc_s, idx_src, idx_dst, rows, heb):
    c = lax.axis_index("c")
    s = lax.axis_index("s")

    nsl = pl.ds(pl.multiple_of(s * NPT, 8), NPT)
    pltpu.sync_copy(hv_hbm.at[c, nsl, :], hv_s.at[nsl, :])
    pltpu.sync_copy(zero_hbm, acc_s.at[nsl, :])
    plsc.subcore_barrier()

    base = s * PER_TILE

    def chunk(ci, carry):
        off = pl.multiple_of(base + ci * K, 8)
        pltpu.sync_copy(src_hbm.at[pl.ds(off, K)], idx_src)
        pltpu.sync_copy(dst_hbm.at[pl.ds(off, K)], idx_dst)
        pltpu.sync_copy(he_hbm.at[c, pl.ds(off, K), :], heb)
        pltpu.sync_copy(hv_s.at[idx_src], rows)

        def mul_row(j, cc):
            for v in range(H // 16):
                sl = pl.ds(v * 16, 16)
                rows[j, sl] = rows[j, sl] * heb[j, sl]
            return cc
        lax.fori_loop(0, K, mul_row, 0)

        pltpu.sync_copy(rows, acc_s.at[idx_dst], add=True)
        return carry

    lax.fori_loop(0, PER_TILE // K, chunk, 0)
    plsc.subcore_barrier()

    pltpu.sync_copy(acc_s.at[nsl, :], out_hbm.at[c, nsl, :])


def _sc_aggregate(hv_halves, he_halves, src, dst, zeros):
    mesh = plsc.VectorSubcoreMesh(core_axis_name="c", subcore_axis_name="s")
    f = pl.kernel(
        _sc_body,
        out_type=jax.ShapeDtypeStruct((2, NP, H), jnp.float32),
        mesh=mesh,
        scratch_types=[
            pltpu.VMEM_SHARED((NP, H), jnp.float32),
            pltpu.VMEM_SHARED((NP, H), jnp.float32),
            pltpu.VMEM((K,), jnp.int32),
            pltpu.VMEM((K,), jnp.int32),
            pltpu.VMEM((K, H), jnp.float32),
            pltpu.VMEM((K, H), jnp.float32),
        ],
        compiler_params=pltpu.CompilerParams(use_tc_tiling_on_sc=False),
    )
    return f(hv_halves, he_halves, src, dst, zeros)


def _out_body(hl_ref, hh_ref, wcl_ref, wch_ref, bc_ref, wo_ref, bo_ref,
              out_ref):
    t = jnp.dot(hl_ref[0], wcl_ref[...], preferred_element_type=jnp.float32)
    t = t + jnp.dot(hh_ref[0], wch_ref[...],
                    preferred_element_type=jnp.float32)
    t = _ssp(t + bc_ref[...])
    o = jnp.dot(t, wo_ref[...], preferred_element_type=jnp.float32)
    out_ref[...] = o + bo_ref[...]


def _project_out(h_halves, w_co, b_co, w_out, b_out):
    BN = 2000
    return pl.pallas_call(
        _out_body,
        grid=(N // BN,),
        in_specs=[
            pl.BlockSpec((1, BN, H), lambda i: (0, i, 0)),
            pl.BlockSpec((1, BN, H), lambda i: (1, i, 0)),
            pl.BlockSpec((H, D), lambda i: (0, 0)),
            pl.BlockSpec((H, D), lambda i: (0, 0)),
            pl.BlockSpec((1, D), lambda i: (0, 0)),
            pl.BlockSpec((D, D), lambda i: (0, 0)),
            pl.BlockSpec((1, D), lambda i: (0, 0)),
        ],
        out_specs=pl.BlockSpec((BN, D), lambda i: (i, 0)),
        out_shape=jax.ShapeDtypeStruct((N, D), jnp.float32),
    )(h_halves, h_halves, w_co[:H], w_co[H:], b_co.reshape(1, D),
      w_out, b_out.reshape(1, D))


def kernel(node_feats, edge_feats, edge_index,
           W_pn, b_pn, W_e1, b_e1, W_e2, b_e2, W_co, b_co, W_out, b_out):
    src = edge_index[0].astype(jnp.int32)
    dst = edge_index[1].astype(jnp.int32)
    pad = jnp.zeros((EP - E,), jnp.int32)
    src_p = jnp.concatenate([src, pad])
    dst_p = jnp.concatenate([dst, pad])
    zeros = jnp.zeros((NPT, H), jnp.float32)

    hv_halves = _hv_halves(node_feats, W_pn, b_pn)
    he_halves = _he_halves(edge_feats, W_e1, b_e1, W_e2, b_e2)
    h_halves = _sc_aggregate(hv_halves, he_halves, src_p, dst_p, zeros)
    return _project_out(h_halves, W_co, b_co, W_out, b_out)


# speedup vs baseline: 2.1839x; 1.3260x over previous
import functools

import jax
import jax.numpy as jnp
from jax import lax
from jax.experimental import pallas as pl
from jax.experimental.pallas import tpu as pltpu
from jax.experimental.pallas import tpu_sc as plsc

N = 10000
NP = 10240
E = 320000
D = 128
H = 64
EIN = 16
NS = 16
K = 128
EP = 327680
PER_TILE = EP // NS
NPT = NP // NS

_LN2 = 0.6931471805599453


def _ssp(x):
    return jax.nn.softplus(x) - _LN2


def _hv_body(x_ref, w_ref, b_ref, out_ref):
    hv = jnp.dot(x_ref[...], w_ref[...], preferred_element_type=jnp.float32)
    out_ref[...] = hv + b_ref[...]


def _hv_proj(x, w, b):
    BN = 2048
    return pl.pallas_call(
        _hv_body,
        grid=(NP // BN,),
        in_specs=[
            pl.BlockSpec((BN, D), lambda i: (i, 0)),
            pl.BlockSpec((D, D), lambda i: (0, 0)),
            pl.BlockSpec((1, D), lambda i: (0, 0)),
        ],
        out_specs=pl.BlockSpec((BN, D), lambda i: (i, 0)),
        out_shape=jax.ShapeDtypeStruct((NP, D), jnp.float32),
    )(x, w, b.reshape(1, D))


def _he_body(ef_ref, w1_ref, b1_ref, w2_ref, b2_ref, out_ref, *, be):
    h1 = jnp.dot(ef_ref[...], w1_ref[...], preferred_element_type=jnp.float32)
    h1 = _ssp(h1 + b1_ref[...])
    h2 = jnp.dot(h1, w2_ref[...], preferred_element_type=jnp.float32)
    h2 = _ssp(h2 + b2_ref[...])
    i = pl.program_id(0)
    row = i * be + lax.broadcasted_iota(jnp.int32, (be, 1), 0)
    out_ref[...] = jnp.where(row < E, h2, 0.0)


def _he_proj(ef, w1, b1, w2, b2):
    BE = 2048
    last = E // BE
    return pl.pallas_call(
        functools.partial(_he_body, be=BE),
        grid=(EP // BE,),
        in_specs=[
            pl.BlockSpec((BE, EIN), lambda i: (jnp.minimum(i, last), 0)),
            pl.BlockSpec((EIN, D), lambda i: (0, 0)),
            pl.BlockSpec((1, D), lambda i: (0, 0)),
            pl.BlockSpec((D, D), lambda i: (0, 0)),
            pl.BlockSpec((1, D), lambda i: (0, 0)),
        ],
        out_specs=pl.BlockSpec((BE, D), lambda i: (i, 0)),
        out_shape=jax.ShapeDtypeStruct((EP, D), jnp.float32),
    )(ef, w1, b1.reshape(1, D), w2, b2.reshape(1, D))


def _sc_body(hv_hbm, he_hbm, src_hbm, dst_hbm, zero_hbm, out_hbm,
             hv_s, acc_s, idx_src, idx_dst, rows, heb):
    c = lax.axis_index("c")
    s = lax.axis_index("s")
    csl = pl.ds(pl.multiple_of(c * H, H), H)

    nsl = pl.ds(pl.multiple_of(s * NPT, 8), NPT)
    pltpu.sync_copy(hv_hbm.at[nsl, csl], hv_s.at[nsl, :])
    pltpu.sync_copy(zero_hbm, acc_s.at[nsl, :])
    plsc.subcore_barrier()

    base = s * PER_TILE

    def chunk(ci, carry):
        off = pl.multiple_of(base + ci * K, 8)
        pltpu.sync_copy(src_hbm.at[pl.ds(off, K)], idx_src)
        pltpu.sync_copy(dst_hbm.at[pl.ds(off, K)], idx_dst)
        pltpu.sync_copy(he_hbm.at[pl.ds(off, K), csl], heb)
        pltpu.sync_copy(hv_s.at[idx_src], rows)

        def mul_row(j, cc):
            for v in range(H // 16):
                sl = pl.ds(v * 16, 16)
                rows[j, sl] = rows[j, sl] * heb[j, sl]
            return cc
        lax.fori_loop(0, K, mul_row, 0)

        pltpu.sync_copy(rows, acc_s.at[idx_dst], add=True)
        return carry

    lax.fori_loop(0, PER_TILE // K, chunk, 0)
    plsc.subcore_barrier()

    pltpu.sync_copy(acc_s.at[nsl, :], out_hbm.at[nsl, csl])


def _sc_aggregate(hv, he, src, dst, zeros):
    mesh = plsc.VectorSubcoreMesh(core_axis_name="c", subcore_axis_name="s")
    f = pl.kernel(
        _sc_body,
        out_type=jax.ShapeDtypeStruct((NP, D), jnp.float32),
        mesh=mesh,
        scratch_types=[
            pltpu.VMEM_SHARED((NP, H), jnp.float32),
            pltpu.VMEM_SHARED((NP, H), jnp.float32),
            pltpu.VMEM((K,), jnp.int32),
            pltpu.VMEM((K,), jnp.int32),
            pltpu.VMEM((K, H), jnp.float32),
            pltpu.VMEM((K, H), jnp.float32),
        ],
        compiler_params=pltpu.CompilerParams(use_tc_tiling_on_sc=False),
    )
    return f(hv, he, src, dst, zeros)


def _out_body(h_ref, wc_ref, bc_ref, wo_ref, bo_ref, out_ref):
    t = jnp.dot(h_ref[...], wc_ref[...], preferred_element_type=jnp.float32)
    t = _ssp(t + bc_ref[...])
    o = jnp.dot(t, wo_ref[...], preferred_element_type=jnp.float32)
    out_ref[...] = o + bo_ref[...]


def _project_out(h, w_co, b_co, w_out, b_out):
    BN = 2000
    return pl.pallas_call(
        _out_body,
        grid=(N // BN,),
        in_specs=[
            pl.BlockSpec((BN, D), lambda i: (i, 0)),
            pl.BlockSpec((D, D), lambda i: (0, 0)),
            pl.BlockSpec((1, D), lambda i: (0, 0)),
            pl.BlockSpec((D, D), lambda i: (0, 0)),
            pl.BlockSpec((1, D), lambda i: (0, 0)),
        ],
        out_specs=pl.BlockSpec((BN, D), lambda i: (i, 0)),
        out_shape=jax.ShapeDtypeStruct((N, D), jnp.float32),
    )(h, w_co, b_co.reshape(1, D), w_out, b_out.reshape(1, D))


def kernel(node_feats, edge_feats, edge_index,
           W_pn, b_pn, W_e1, b_e1, W_e2, b_e2, W_co, b_co, W_out, b_out):
    src = edge_index[0].astype(jnp.int32)
    dst = edge_index[1].astype(jnp.int32)
    pad = jnp.zeros((EP - E,), jnp.int32)
    src_p = jnp.concatenate([src, pad])
    dst_p = jnp.concatenate([dst, pad])
    zeros = jnp.zeros((NPT, H), jnp.float32)

    hv = _hv_proj(node_feats, W_pn, b_pn)
    he = _he_proj(edge_feats, W_e1, b_e1, W_e2, b_e2)
    h = _sc_aggregate(hv, he, src_p, dst_p, zeros)
    return _project_out(h, W_co, b_co, W_out, b_out)
